# baseline (speedup 1.0000x reference)
import functools

import jax
import jax.numpy as jnp
from jax import lax
from jax.experimental import pallas as pl
from jax.experimental.pallas import tpu as pltpu
from jax.experimental.pallas import tpu_sc as plsc

_K = 5
_L = 16
_NCAND = _K * _L


def _tc_d_body(fc_ref, fs_ref, d_ref):
    f32 = jnp.float32
    fc = fc_ref[0]
    fs = fs_ref[0]
    m, F = fc.shape
    eps = f32(1e-8)

    hi = jax.lax.Precision.HIGHEST
    ones_row_F = jnp.ones((1, F), f32)
    nc_col = jnp.sum(fc * fc, axis=1, keepdims=True)
    nc_row = jax.lax.dot_general(ones_row_F, fc * fc, (((1,), (1,)), ((), ())),
                                 precision=hi, preferred_element_type=f32)
    ns_row = jax.lax.dot_general(ones_row_F, fs * fs, (((1,), (1,)), ((), ())),
                                 precision=hi, preferred_element_type=f32)

    def ncc(x, x_norm_row):
        g = jax.lax.dot_general(fc, x, (((1,), (1,)), ((), ())),
                                preferred_element_type=f32)
        return (g + eps) / (jnp.sqrt(nc_col * x_norm_row) + eps)

    d_ref[0, pl.ds(0, m), :] = ncc(fc, nc_row)
    d_ref[0, pl.ds(m, m), :] = ncc(fs, ns_row)


def _sc_topcand_body(d_hbm, cv_hbm, ci_hbm, buf, oval, oidx):
    info = plsc.get_sparse_core_info()
    nw = info.num_cores * info.num_subcores
    wid = lax.axis_index("s") * info.num_cores + lax.axis_index("c")
    ch = buf.shape[0]
    n = buf.shape[1]
    n_chunks = cv_hbm.shape[0] // (nw * ch)
    base = wid * (n_chunks * ch)
    lane = lax.broadcasted_iota(jnp.int32, (_L,), 0)
    neg_inf = jnp.full((_L,), -jnp.inf, jnp.float32)
    zero_i = jnp.zeros((_L,), jnp.int32)

    def chunk_body(c, _):
        rb = base + c * ch
        pltpu.sync_copy(d_hbm.at[pl.ds(rb, ch)], buf)

        def row_body(r, _2):
            v = [neg_inf] * _K
            vi = [zero_i] * _K
            for blk in range(n // _L):
                x = buf[r, pl.ds(blk * _L, _L)]
                xi = lane + blk * _L
                for t in range(_K):
                    mgt = x > v[t]
                    v[t], x = jnp.where(mgt, x, v[t]), jnp.where(mgt, v[t], x)
                    vi[t], xi = jnp.where(mgt, xi, vi[t]), jnp.where(mgt, vi[t], xi)
            for t in range(_K):
                oval[r, pl.ds(t * _L, _L)] = v[t]
                oidx[r, pl.ds(t * _L, _L)] = vi[t]
            return 0

        lax.fori_loop(0, ch, row_body, 0)
        pltpu.sync_copy(oval, cv_hbm.at[pl.ds(rb, ch)])
        pltpu.sync_copy(oidx, ci_hbm.at[pl.ds(rb, ch)])
        return 0

    lax.fori_loop(0, n_chunks, chunk_body, 0)


def _sc_topcand(d_flat):
    rows, n = d_flat.shape
    ch = 16
    mesh = plsc.VectorSubcoreMesh(core_axis_name="c", subcore_axis_name="s")
    fn = pl.kernel(
        _sc_topcand_body,
        out_type=[
            jax.ShapeDtypeStruct((rows, _NCAND), jnp.float32),
            jax.ShapeDtypeStruct((rows, _NCAND), jnp.int32),
        ],
        mesh=mesh,
        scratch_types=[
            pltpu.VMEM((ch, n), jnp.float32),
            pltpu.VMEM((ch, _NCAND), jnp.float32),
            pltpu.VMEM((ch, _NCAND), jnp.int32),
        ],
    )
    return fn(d_flat)


def _tc_agg_body(fc_ref, fs_ref, cv_ref, ci_ref, W1_ref, b1_ref, W2_ref,
                 b2_ref, out_ref):
    f32 = jnp.float32
    fc = fc_ref[0]
    fs = fs_ref[0]
    m, F = fc.shape
    n = fs.shape[0]
    val = cv_ref[0]
    ci = ci_ref[0]

    iota_s = lax.broadcasted_iota(jnp.int32, (2 * m, n), 1)
    big = jnp.int32(1 << 30)
    S = jnp.zeros((2 * m, n), f32)
    for _ in range(_K):
        mx = jnp.max(val, axis=1, keepdims=True)
        ismx = val == mx
        am = jnp.min(jnp.where(ismx, ci, big), axis=1, keepdims=True)
        val = jnp.where(ismx & (ci == am), -jnp.inf, val)
        S = S + (iota_s == am).astype(f32)

    S2 = S[:m, :]
    S1 = S[m:, :]

    ones_row_m = jnp.ones((1, m), f32)
    cnt2 = jax.lax.dot_general(ones_row_m, S2, (((1,), (0,)), ((), ())),
                               preferred_element_type=f32)
    cnt1 = jax.lax.dot_general(ones_row_m, S1, (((1,), (0,)), ((), ())),
                               preferred_element_type=f32)
    S2w = S2 * jax.lax.rsqrt(jnp.maximum(cnt2, 1.0))
    S1w = S1 * jax.lax.rsqrt(jnp.maximum(cnt1, 1.0))

    c_in = f32((2.0 * _K) ** -0.5)
    W1 = W1_ref[...]
    W2 = W2_ref[...]
    b1 = b1_ref[...]
    b2 = b2_ref[...]

    def mm(a, b):
        return jax.lax.dot_general(a, b, (((1,), (0,)), ((), ())),
                                   preferred_element_type=f32)

    agg1 = (mm(S2w, fc) + mm(S1w, fs)) * c_in
    h1 = jnp.maximum(mm(agg1, W1) + b1, 0.0)
    h1s = jnp.maximum(b1, 0.0)

    rs1 = mm(S1w, jnp.ones((n, 1), f32))
    agg2 = (mm(S2w, h1) + rs1 * h1s) * c_in
    out_ref[0] = mm(agg2, W2) + b2


def kernel(ys, yc, W1, b1, W2, b2):
    B, N1, N2, F = ys.shape
    _, C, P, _, M1, M2 = yc.shape
    n = N1 * N2
    m = M1 * M2
    fs = ys.reshape(B, n, F)
    fc = jnp.transpose(yc, (0, 4, 5, 1, 2, 3)).reshape(B, m, F)
    b1r = b1.reshape(1, F)
    b2r = b2.reshape(1, F)

    d_all = pl.pallas_call(
        _tc_d_body,
        grid=(B,),
        in_specs=[
            pl.BlockSpec((1, m, F), lambda b: (b, 0, 0)),
            pl.BlockSpec((1, n, F), lambda b: (b, 0, 0)),
        ],
        out_specs=pl.BlockSpec((1, 2 * m, n), lambda b: (b, 0, 0)),
        out_shape=jax.ShapeDtypeStruct((B, 2 * m, n), jnp.float32),
    )(fc, fs)

    cv, ci = _sc_topcand(d_all.reshape(B * 2 * m, n))

    out = pl.pallas_call(
        _tc_agg_body,
        grid=(B,),
        in_specs=[
            pl.BlockSpec((1, m, F), lambda b: (b, 0, 0)),
            pl.BlockSpec((1, n, F), lambda b: (b, 0, 0)),
            pl.BlockSpec((1, 2 * m, _NCAND), lambda b: (b, 0, 0)),
            pl.BlockSpec((1, 2 * m, _NCAND), lambda b: (b, 0, 0)),
            pl.BlockSpec((F, F), lambda b: (0, 0)),
            pl.BlockSpec((1, F), lambda b: (0, 0)),
            pl.BlockSpec((F, F), lambda b: (0, 0)),
            pl.BlockSpec((1, F), lambda b: (0, 0)),
        ],
        out_specs=pl.BlockSpec((1, m, F), lambda b: (b, 0, 0)),
        out_shape=jax.ShapeDtypeStruct((B, m, F), jnp.float32),
    )(fc, fs, cv.reshape(B, 2 * m, _NCAND), ci.reshape(B, 2 * m, _NCAND),
      W1, b1r, W2, b2r)

    return jnp.transpose(out, (0, 2, 1)).reshape(B, C, P, P, M1, M2)
